# baseline (speedup 1.0000x reference)
import functools

import jax
import jax.numpy as jnp
from jax import lax
from jax.experimental import pallas as pl
from jax.experimental.pallas import tpu as pltpu
from jax.experimental.pallas import tpu_sc as plsc

NUM_WORKERS = 32
SEG0 = 128
SEG1 = 72
NBUF = 4


def _make_gather(n_rows, seq, d):
    rows_per_w = n_rows // NUM_WORKERS
    mesh = plsc.VectorSubcoreMesh(core_axis_name="c", subcore_axis_name="s")
    segs = [(0, SEG0), (SEG0, SEG1)]

    @functools.partial(
        pl.kernel,
        out_type=jax.ShapeDtypeStruct((n_rows, seq, d), jnp.float32),
        mesh=mesh,
        scratch_types=(
            [pltpu.VMEM((rows_per_w, seq), jnp.int32),
             pltpu.VMEM((NBUF, SEG0, d), jnp.float32)]
            + [pltpu.SemaphoreType.DMA] * (2 * NBUF)
        ),
        compiler_params=pltpu.CompilerParams(use_tc_tiling_on_sc=False),
    )
    def gather_kernel(idx_hbm, table_hbm, out_hbm, idx_v, rows_v, *sems):
        gsem = sems[:NBUF]
        wsem = sems[NBUF:]
        wid = lax.axis_index("s") * 2 + lax.axis_index("c")
        row0 = wid * rows_per_w
        pltpu.sync_copy(idx_hbm.at[pl.ds(row0, rows_per_w)], idx_v)

        def fire(r, b, off, npull):
            pltpu.async_copy(
                table_hbm.at[idx_v.at[r, pl.ds(off, npull)]],
                rows_v.at[b, pl.ds(0, npull)], gsem[b])

        def drain_g(b, npull):
            pltpu.make_async_copy(table_hbm.at[pl.ds(0, npull)],
                                  rows_v.at[b, pl.ds(0, npull)],
                                  gsem[b]).wait()

        def put(r, b, off, npull):
            pltpu.async_copy(rows_v.at[b, pl.ds(0, npull)],
                             out_hbm.at[row0 + r, pl.ds(off, npull)],
                             wsem[b])

        def drain_w(b, npull):
            pltpu.make_async_copy(table_hbm.at[pl.ds(0, npull)],
                                  rows_v.at[b, pl.ds(0, npull)],
                                  wsem[b]).wait()

        fire(0, 0, 0, SEG0)
        fire(0, 1, SEG0, SEG1)
        fire(1, 2, 0, SEG0)

        n_pairs = rows_per_w // 2
        assert rows_per_w % 2 == 0

        def outer(p, carry):
            for k in range(4):
                c = 4 * p + k
                r = 2 * p + (k // 2)
                off, npull = segs[k % 2]
                b = k
                fb = (k + 3) % 4
                foff, fnpull = segs[(k + 3) % 2]

                @pl.when(c >= 1)
                def _():
                    drain_w(fb, segs[(k + 1) % 2][1])

                @pl.when(c + 3 < 2 * rows_per_w)
                def _():
                    fire(r + (k + 3) // 2 - k // 2, fb, foff, fnpull)

                drain_g(b, npull)
                put(r, b, off, npull)
            return carry

        lax.fori_loop(0, n_pairs, outer, 0, unroll=False)
        drain_w((2 * rows_per_w - 1) % NBUF, SEG1)

    return gather_kernel


def kernel(code, embedding_table):
    b, s = code.shape
    v, d = embedding_table.shape
    assert b % NUM_WORKERS == 0 and s == SEG0 + SEG1
    out = _make_gather(b, s, d)(code.astype(jnp.int32), embedding_table)
    return out
